# baseline (speedup 1.0000x reference)
import functools

import jax
import jax.numpy as jnp
from jax import lax
from jax.experimental import pallas as pl
from jax.experimental.pallas import tpu as pltpu
from jax.experimental.pallas import tpu_sc as plsc

NUM_FIELDS = 26
VOCAB = 100000
EMBED_DIM = 8
BATCH = 4096

NC, NS = 2, 16
NW = NC * NS
B_PER_W = BATCH // NW
NBUF = 4
WPP = VOCAB // 2


def _sc_body(idx_hbm, tab_hbm, out_hbm, idx_v, vh_v, g_v, acc_v, sem):
    cid = lax.axis_index("c")
    sid = lax.axis_index("s")
    wid = sid * NC + cid
    bbase = wid * B_PER_W

    pltpu.sync_copy(idx_hbm.at[:, pl.ds(bbase, B_PER_W)], idx_v)
    for f in range(NUM_FIELDS):
        for c in range(B_PER_W // 16):
            v = idx_v[f, pl.ds(c * 16, 16)]
            vh_v[f, pl.ds(c * 16, 16)] = lax.shift_right_logical(v, 1)

    def fire(f, buf):
        return [
            pltpu.async_copy(
                tab_hbm.at[pl.ds((f * EMBED_DIM + d) * WPP, WPP)]
                .at[vh_v.at[f]],
                g_v.at[buf, d],
                sem,
            )
            for d in range(EMBED_DIM)
        ]

    hi_mask = jnp.full((16,), -65536, jnp.int32)
    pending = [fire(f, f % NBUF) for f in range(NBUF - 1)]
    for f in range(NUM_FIELDS):
        if f + NBUF - 1 < NUM_FIELDS:
            pending.append(fire(f + NBUF - 1, (f + NBUF - 1) % NBUF))
        for cp in pending.pop(0):
            cp.wait()
        buf = f % NBUF
        for c in range(B_PER_W // 16):
            odd = (idx_v[f, pl.ds(c * 16, 16)] & 1) == 1
            tot = None
            for d in range(EMBED_DIM):
                g = g_v[buf, d, pl.ds(c * 16, 16)]
                val = plsc.bitcast(
                    jnp.where(odd, g & hi_mask, lax.shift_left(g, 16)),
                    jnp.float32,
                )
                tot = val if tot is None else tot + val
            if f == 0:
                acc_v[pl.ds(c * 16, 16)] = tot
            else:
                acc_v[pl.ds(c * 16, 16)] = acc_v[pl.ds(c * 16, 16)] + tot

    for c in range(B_PER_W // 16):
        x = acc_v[pl.ds(c * 16, 16)]
        acc_v[pl.ds(c * 16, 16)] = 1.0 / (1.0 + jnp.exp(-x))
    pltpu.sync_copy(acc_v, out_hbm.at[pl.ds(bbase, B_PER_W)])


@functools.partial(
    pl.kernel,
    out_type=jax.ShapeDtypeStruct((BATCH,), jnp.float32),
    mesh=plsc.VectorSubcoreMesh(
        core_axis_name="c", subcore_axis_name="s", num_cores=NC, num_subcores=NS
    ),
    scratch_types=[
        pltpu.VMEM((NUM_FIELDS, B_PER_W), jnp.int32),
        pltpu.VMEM((NUM_FIELDS, B_PER_W), jnp.int32),
        pltpu.VMEM((NBUF, EMBED_DIM, B_PER_W), jnp.int32),
        pltpu.VMEM((B_PER_W,), jnp.float32),
        pltpu.SemaphoreType.DMA,
    ],
    compiler_params=pltpu.CompilerParams(
        use_tc_tiling_on_sc=False, needs_layout_passes=False
    ),
)
def _sc_kernel(idx_hbm, tab_hbm, out_hbm, idx_v, vh_v, g_v, acc_v, sem):
    _sc_body(idx_hbm, tab_hbm, out_hbm, idx_v, vh_v, g_v, acc_v, sem)


def kernel(indices, table):
    idxT = indices.T
    tab_bf = table.transpose(0, 2, 1).astype(jnp.bfloat16)
    tab_i32 = lax.bitcast_convert_type(
        tab_bf.reshape(-1, 2), jnp.int32
    )
    out = _sc_kernel(idxT, tab_i32)
    return out.reshape(BATCH, 1)


# speedup vs baseline: 45.5167x; 45.5167x over previous
import functools

import jax
import jax.numpy as jnp
from jax import lax
from jax.experimental import pallas as pl
from jax.experimental.pallas import tpu as pltpu
from jax.experimental.pallas import tpu_sc as plsc

NUM_FIELDS = 26
VOCAB = 100000
EMBED_DIM = 8
BATCH = 4096

NC, NS = 2, 16
NW = NC * NS
B_PER_W = BATCH // NW
NBUF = 4


def _sc_body(idx_hbm, tab_hbm, out_hbm, idx_v, g_v, acc_v, sem):
    cid = lax.axis_index("c")
    sid = lax.axis_index("s")
    wid = sid * NC + cid
    bbase = wid * B_PER_W

    pltpu.sync_copy(idx_hbm.at[:, pl.ds(bbase, B_PER_W)], idx_v)

    def fire(f, buf):
        return [
            pltpu.async_copy(
                tab_hbm.at[pl.ds((f * EMBED_DIM + d) * VOCAB, VOCAB)]
                .at[idx_v.at[f]],
                g_v.at[buf, d],
                sem,
            )
            for d in range(EMBED_DIM)
        ]

    pending = [fire(f, f % NBUF) for f in range(NBUF - 1)]
    for f in range(NUM_FIELDS):
        if f + NBUF - 1 < NUM_FIELDS:
            pending.append(fire(f + NBUF - 1, (f + NBUF - 1) % NBUF))
        for cp in pending.pop(0):
            cp.wait()
        buf = f % NBUF
        for c in range(B_PER_W // 16):
            tot = g_v[buf, 0, pl.ds(c * 16, 16)]
            for d in range(1, EMBED_DIM):
                tot = tot + g_v[buf, d, pl.ds(c * 16, 16)]
            if f == 0:
                acc_v[pl.ds(c * 16, 16)] = tot
            else:
                acc_v[pl.ds(c * 16, 16)] = acc_v[pl.ds(c * 16, 16)] + tot

    for c in range(B_PER_W // 16):
        x = acc_v[pl.ds(c * 16, 16)]
        acc_v[pl.ds(c * 16, 16)] = 1.0 / (1.0 + jnp.exp(-x))
    pltpu.sync_copy(acc_v, out_hbm.at[pl.ds(bbase, B_PER_W)])


@functools.partial(
    pl.kernel,
    out_type=jax.ShapeDtypeStruct((BATCH,), jnp.float32),
    mesh=plsc.VectorSubcoreMesh(
        core_axis_name="c", subcore_axis_name="s", num_cores=NC, num_subcores=NS
    ),
    scratch_types=[
        pltpu.VMEM((NUM_FIELDS, B_PER_W), jnp.int32),
        pltpu.VMEM((NBUF, EMBED_DIM, B_PER_W), jnp.float32),
        pltpu.VMEM((B_PER_W,), jnp.float32),
        pltpu.SemaphoreType.DMA,
    ],
    compiler_params=pltpu.CompilerParams(
        use_tc_tiling_on_sc=False, needs_layout_passes=False
    ),
)
def _sc_kernel(idx_hbm, tab_hbm, out_hbm, idx_v, g_v, acc_v, sem):
    _sc_body(idx_hbm, tab_hbm, out_hbm, idx_v, g_v, acc_v, sem)


def kernel(indices, table):
    idxT = indices.T
    tab_flat = table.transpose(0, 2, 1).reshape(-1)
    out = _sc_kernel(idxT, tab_flat)
    return out.reshape(BATCH, 1)


# speedup vs baseline: 45.8314x; 1.0069x over previous
import functools

import jax
import jax.numpy as jnp
from jax import lax
from jax.experimental import pallas as pl
from jax.experimental.pallas import tpu as pltpu
from jax.experimental.pallas import tpu_sc as plsc

NUM_FIELDS = 26
VOCAB = 100000
EMBED_DIM = 8
BATCH = 4096

NC, NS = 2, 16
NW = NC * NS
B_PER_W = BATCH // NW
NBUF = 6


def _sc_body(idx_hbm, tab_hbm, out_hbm, idx_v, g_v, acc_v, sem):
    cid = lax.axis_index("c")
    sid = lax.axis_index("s")
    wid = sid * NC + cid
    bbase = wid * B_PER_W

    pltpu.sync_copy(idx_hbm.at[:, pl.ds(bbase, B_PER_W)], idx_v)

    def fire(f, buf):
        return [
            pltpu.async_copy(
                tab_hbm.at[pl.ds((f * EMBED_DIM + d) * VOCAB, VOCAB)]
                .at[idx_v.at[f]],
                g_v.at[buf, d],
                sem,
            )
            for d in range(EMBED_DIM)
        ]

    pending = [fire(f, f % NBUF) for f in range(NBUF - 1)]
    for f in range(NUM_FIELDS):
        if f + NBUF - 1 < NUM_FIELDS:
            pending.append(fire(f + NBUF - 1, (f + NBUF - 1) % NBUF))
        for cp in pending.pop(0):
            cp.wait()
        buf = f % NBUF
        for c in range(B_PER_W // 16):
            tot = g_v[buf, 0, pl.ds(c * 16, 16)]
            for d in range(1, EMBED_DIM):
                tot = tot + g_v[buf, d, pl.ds(c * 16, 16)]
            if f == 0:
                acc_v[pl.ds(c * 16, 16)] = tot
            else:
                acc_v[pl.ds(c * 16, 16)] = acc_v[pl.ds(c * 16, 16)] + tot

    for c in range(B_PER_W // 16):
        x = acc_v[pl.ds(c * 16, 16)]
        acc_v[pl.ds(c * 16, 16)] = 1.0 / (1.0 + jnp.exp(-x))
    pltpu.sync_copy(acc_v, out_hbm.at[pl.ds(bbase, B_PER_W)])


@functools.partial(
    pl.kernel,
    out_type=jax.ShapeDtypeStruct((BATCH,), jnp.float32),
    mesh=plsc.VectorSubcoreMesh(
        core_axis_name="c", subcore_axis_name="s", num_cores=NC, num_subcores=NS
    ),
    scratch_types=[
        pltpu.VMEM((NUM_FIELDS, B_PER_W), jnp.int32),
        pltpu.VMEM((NBUF, EMBED_DIM, B_PER_W), jnp.float32),
        pltpu.VMEM((B_PER_W,), jnp.float32),
        pltpu.SemaphoreType.DMA,
    ],
    compiler_params=pltpu.CompilerParams(
        use_tc_tiling_on_sc=False, needs_layout_passes=False
    ),
)
def _sc_kernel(idx_hbm, tab_hbm, out_hbm, idx_v, g_v, acc_v, sem):
    _sc_body(idx_hbm, tab_hbm, out_hbm, idx_v, g_v, acc_v, sem)


def kernel(indices, table):
    idxT = indices.T
    tab_flat = table.transpose(0, 2, 1).reshape(-1)
    out = _sc_kernel(idxT, tab_flat)
    return out.reshape(BATCH, 1)
